# Initial kernel scaffold; baseline (speedup 1.0000x reference)
import jax
import jax.numpy as jnp
from jax.experimental import pallas as pl


def kernel(h_user, h_item, edge_index_clicks, edge_index_clicked_by, W_clicks, b_clicks, W_clicked_by, b_clicked_by, W_self_user, b_self_user, W_self_item, b_self_item):
    raise NotImplementedError("write your pallas kernel here")


# speedup vs baseline: 2.4632x; 2.4632x over previous
import functools

import jax
import jax.numpy as jnp
from jax import lax
from jax.experimental import pallas as pl
from jax.experimental.pallas import tpu as pltpu
from jax.experimental.pallas import tpu_sc as plsc

N = 50000
E = 500000
D = 128
S = 32
NSLAB = D // S
TILES = 16
EB = 128
CB = 7
NC = 35
NB = NC * CB
EPT = NB * EB
E_PAD = TILES * EPT
NP = 50176
RPT = NP // TILES
ZR = 224
NZ = RPT // ZR
CW = 8
ZC = 448
NZC = RPT // ZC


def _prep_edges(ei):
    src = ei[0].astype(jnp.int32)
    dst = ei[1].astype(jnp.int32)
    npad = E_PAD - E
    ar = jnp.arange(npad, dtype=jnp.int32)
    src_p = jnp.concatenate([src, ar % N])
    dst_p = jnp.concatenate([dst, N + (ar % (NP - N))])
    return (src_p.reshape(TILES, NC, CB, EB),
            dst_p.reshape(TILES, NC, CB, EB))


def _slabs(h):
    h4 = jnp.transpose(h.reshape(N, NSLAB, S), (1, 0, 2))
    return tuple(h4[s] for s in range(NSLAB))


_MESH = plsc.VectorSubcoreMesh(core_axis_name="c", subcore_axis_name="s")
_SC_PARAMS = pltpu.CompilerParams(use_tc_tiling_on_sc=False)


def _sc_aggregate(hu, hi, src_c, dst_c, src_b, dst_b):
    zero_blk = jnp.zeros((ZR, S), jnp.float32)
    acc_ty = jax.ShapeDtypeStruct((NP, S), jnp.float32)

    @functools.partial(
        pl.kernel,
        mesh=_MESH,
        out_type=[acc_ty] * (2 * NSLAB),
        scratch_types=[
            pltpu.VMEM((CB, EB), jnp.int32),
            pltpu.VMEM((CB, EB), jnp.int32),
            pltpu.VMEM((EB, S), jnp.float32),
            pltpu.VMEM((ZR, S), jnp.float32),
            pltpu.VMEM_SHARED((NP, S), jnp.float32),
            pltpu.SemaphoreType.DMA,
        ],
        compiler_params=_SC_PARAMS,
    )
    def sc_agg(*refs):
        hu_t = refs[0:NSLAB]
        hi_t = refs[NSLAB:2 * NSLAB]
        srcc_h, dstc_h, srcb_h, dstb_h, zb_h = refs[2 * NSLAB:2 * NSLAB + 5]
        o = 2 * NSLAB + 5
        ai_t = refs[o:o + NSLAB]
        au_t = refs[o + NSLAB:o + 2 * NSLAB]
        src_v, dst_v, rows_v, wb_v, acc_sp, sem = refs[o + 2 * NSLAB:]
        c = lax.axis_index("c")
        t = lax.axis_index("s")
        r0 = t * RPT

        def process(tables, src_h, dst_h, accs):
            for slab in range(NSLAB):
                pltpu.sync_copy(zb_h, wb_v)
                for j in range(NZ):
                    pltpu.sync_copy(wb_v, acc_sp.at[pl.ds(r0 + j * ZR, ZR)])
                plsc.subcore_barrier()

                def chunk(ci_, carry):
                    pltpu.sync_copy(src_h.at[t, ci_], src_v)
                    pltpu.sync_copy(dst_h.at[t, ci_], dst_v)
                    for b in range(CB):
                        pltpu.async_copy(
                            tables[slab].at[src_v.at[b]], rows_v, sem
                        ).wait()
                        pltpu.sync_copy(
                            rows_v, acc_sp.at[dst_v.at[b]], add=True)
                    return carry

                lax.fori_loop(0, NC, chunk, 0)
                plsc.subcore_barrier()
                for j in range(NZ):
                    rr = r0 + j * ZR
                    pltpu.sync_copy(acc_sp.at[pl.ds(rr, ZR)], wb_v)
                    pltpu.sync_copy(wb_v, accs[slab].at[pl.ds(rr, ZR)])

        @pl.when(c == 0)
        def _():
            process(hu_t, srcc_h, dstc_h, ai_t)

        @pl.when(c == 1)
        def _():
            process(hi_t, srcb_h, dstb_h, au_t)

    outs = sc_agg(*hu, *hi, src_c, dst_c, src_b, dst_b, zero_blk)
    return outs[0:NSLAB], outs[NSLAB:2 * NSLAB]


def _sc_count(dst_c, dst_b):
    zero_blk = jnp.zeros((ZC, CW), jnp.float32)
    ones_blk = jnp.ones((EB, CW), jnp.float32)
    cnt_ty = jax.ShapeDtypeStruct((NP, CW), jnp.float32)

    @functools.partial(
        pl.kernel,
        mesh=_MESH,
        out_type=[cnt_ty, cnt_ty],
        scratch_types=[
            pltpu.VMEM((CB, EB), jnp.int32),
            pltpu.VMEM((EB, CW), jnp.float32),
            pltpu.VMEM((ZC, CW), jnp.float32),
            pltpu.VMEM_SHARED((NP, CW), jnp.float32),
        ],
        compiler_params=_SC_PARAMS,
    )
    def sc_cnt(dstc_h, dstb_h, z_h, ones_h, co_i, co_u,
               dst_v, ones_v, z_v, cnt_sp):
        c = lax.axis_index("c")
        t = lax.axis_index("s")
        r0 = t * RPT

        def process(dst_h, cnt):
            pltpu.sync_copy(ones_h, ones_v)
            pltpu.sync_copy(z_h, z_v)
            for j in range(NZC):
                pltpu.sync_copy(z_v, cnt_sp.at[pl.ds(r0 + j * ZC, ZC)])
            plsc.subcore_barrier()

            def chunk(ci_, carry):
                pltpu.sync_copy(dst_h.at[t, ci_], dst_v)
                for b in range(CB):
                    pltpu.sync_copy(
                        ones_v, cnt_sp.at[dst_v.at[b]], add=True)
                return carry

            lax.fori_loop(0, NC, chunk, 0)
            plsc.subcore_barrier()
            for j in range(NZC):
                rr = r0 + j * ZC
                pltpu.sync_copy(cnt_sp.at[pl.ds(rr, ZC)], z_v)
                pltpu.sync_copy(z_v, cnt.at[pl.ds(rr, ZC)])

        @pl.when(c == 0)
        def _():
            process(dstc_h, co_i)

        @pl.when(c == 1)
        def _():
            process(dstb_h, co_u)

    return sc_cnt(dst_c, dst_b, zero_blk, ones_blk)


def _tc_combine(h_pad, accs, cnt, W_n, b_n, W_s, b_s):
    BLK = 512
    grid = (NP // BLK,)

    def body(*refs):
        h_ref = refs[0]
        a_refs = refs[1:1 + NSLAB]
        cnt_ref, wn_ref, bn_ref, ws_ref, bs_ref, o_ref = refs[1 + NSLAB:]
        cntv = cnt_ref[...][:, 0:1]
        inv = 1.0 / jnp.maximum(cntv, 1.0)
        a = jnp.concatenate([r[...] for r in a_refs], axis=1) * inv
        neigh = lax.dot_general(a, wn_ref[...], (((1,), (1,)), ((), ())),
                                preferred_element_type=jnp.float32)
        selfv = lax.dot_general(h_ref[...], ws_ref[...],
                                (((1,), (1,)), ((), ())),
                                preferred_element_type=jnp.float32)
        mask = (cntv > 0.0).astype(jnp.float32)
        o_ref[...] = neigh + mask * bn_ref[...] + selfv + bs_ref[...]

    row_spec = lambda w: pl.BlockSpec((BLK, w), lambda i: (i, 0))
    full_spec = lambda r, c: pl.BlockSpec((r, c), lambda i: (0, 0))
    return pl.pallas_call(
        body,
        grid=grid,
        in_specs=[row_spec(D)] + [row_spec(S)] * NSLAB + [row_spec(CW)]
        + [full_spec(D, D), full_spec(1, D), full_spec(D, D), full_spec(1, D)],
        out_specs=row_spec(D),
        out_shape=jax.ShapeDtypeStruct((NP, D), jnp.float32),
    )(h_pad, *accs, cnt, W_n, b_n, W_s, b_s)


def kernel(h_user, h_item, edge_index_clicks, edge_index_clicked_by,
           W_clicks, b_clicks, W_clicked_by, b_clicked_by,
           W_self_user, b_self_user, W_self_item, b_self_item):
    src_c, dst_c = _prep_edges(edge_index_clicks)
    src_b, dst_b = _prep_edges(edge_index_clicked_by)
    hu = _slabs(h_user)
    hi = _slabs(h_item)

    acc_item, acc_user = _sc_aggregate(hu, hi, src_c, dst_c, src_b, dst_b)
    cnt_item, cnt_user = _sc_count(dst_c, dst_b)

    pad = jnp.zeros((NP - N, D), jnp.float32)
    h_user_pad = jnp.concatenate([h_user, pad])
    h_item_pad = jnp.concatenate([h_item, pad])

    out_user = _tc_combine(h_user_pad, acc_user, cnt_user,
                           W_clicked_by, b_clicked_by.reshape(1, D),
                           W_self_user, b_self_user.reshape(1, D))
    out_item = _tc_combine(h_item_pad, acc_item, cnt_item,
                           W_clicks, b_clicks.reshape(1, D),
                           W_self_item, b_self_item.reshape(1, D))
    return (out_user[:N], out_item[:N])


# speedup vs baseline: 3.6775x; 1.4930x over previous
import functools

import jax
import jax.numpy as jnp
from jax import lax
from jax.experimental import pallas as pl
from jax.experimental.pallas import tpu as pltpu
from jax.experimental.pallas import tpu_sc as plsc

N = 50000
E = 500000
D = 128
S = 32
NSLAB = D // S
TILES = 16
EB = 128
GB = 4
BPM = 32
MC = 8
MGROUPS = BPM // GB
NB = MC * BPM
EPT = NB * EB
E_PAD = TILES * EPT
CB = 7
NC = 35
NBC = NC * CB
EPTC = NBC * EB
EC_PAD = TILES * EPTC
NP = 50176
RPT = NP // TILES
ZR = 112
NZ = RPT // ZR
CW = 8
ZC = 448
NZC = RPT // ZC


def _prep_edges(ei, e_pad, shape):
    src = ei[0].astype(jnp.int32)
    dst = ei[1].astype(jnp.int32)
    npad = e_pad - E
    ar = jnp.arange(npad, dtype=jnp.int32)
    src_p = jnp.concatenate([src, ar % N])
    dst_p = jnp.concatenate([dst, N + (ar % (NP - N))])
    return src_p.reshape(shape), dst_p.reshape(shape)


def _slabs(h):
    h4 = jnp.transpose(h.reshape(N, NSLAB, S), (1, 0, 2))
    return tuple(h4[s] for s in range(NSLAB))


_MESH = plsc.VectorSubcoreMesh(core_axis_name="c", subcore_axis_name="s")
_SC_PARAMS = pltpu.CompilerParams(use_tc_tiling_on_sc=False)


def _sc_aggregate(hu, hi, src_c, dst_c, src_b, dst_b):
    zero_blk = jnp.zeros((ZR, S), jnp.float32)
    acc_ty = jax.ShapeDtypeStruct((NP, S), jnp.float32)

    @functools.partial(
        pl.kernel,
        mesh=_MESH,
        out_type=[acc_ty] * (2 * NSLAB),
        scratch_types=[
            pltpu.VMEM((BPM, EB), jnp.int32),
            pltpu.VMEM((BPM, EB), jnp.int32),
            pltpu.VMEM((GB, EB, S), jnp.float32),
            pltpu.VMEM((ZR, S), jnp.float32),
            pltpu.VMEM_SHARED((NP, S), jnp.float32),
            pltpu.SemaphoreType.DMA,
            pltpu.SemaphoreType.DMA,
        ],
        compiler_params=_SC_PARAMS,
    )
    def sc_agg(*refs):
        hu_t = refs[0:NSLAB]
        hi_t = refs[NSLAB:2 * NSLAB]
        srcc_h, dstc_h, srcb_h, dstb_h, zb_h = refs[2 * NSLAB:2 * NSLAB + 5]
        o = 2 * NSLAB + 5
        ai_t = refs[o:o + NSLAB]
        au_t = refs[o + NSLAB:o + 2 * NSLAB]
        (src_v, dst_v, rows_v, wb_v, acc_sp,
         gsem, ssem) = refs[o + 2 * NSLAB:]
        c = lax.axis_index("c")
        t = lax.axis_index("s")
        r0 = t * RPT

        def drain_scatters():
            for i in range(GB):
                pltpu.make_async_copy(
                    rows_v.at[i], acc_sp.at[dst_v.at[0]], ssem).wait()

        def process(tables, src_h, dst_h, accs):
            for slab in range(NSLAB):
                pltpu.sync_copy(zb_h, wb_v)

                def zero_j(j, carry):
                    pltpu.sync_copy(wb_v, acc_sp.at[pl.ds(r0 + j * ZR, ZR)])
                    return carry

                lax.fori_loop(0, NZ, zero_j, 0)
                plsc.subcore_barrier()

                def group(g, carry):
                    @pl.when(g > 0)
                    def _():
                        drain_scatters()

                    @pl.when(lax.rem(g, MGROUPS) == 0)
                    def _():
                        m = lax.div(g, MGROUPS)
                        pltpu.sync_copy(src_h.at[t, m], src_v)
                        pltpu.sync_copy(dst_h.at[t, m], dst_v)

                    b0 = lax.rem(g, MGROUPS) * GB
                    gds = []
                    for i in range(GB):
                        gds.append(pltpu.async_copy(
                            tables[slab].at[src_v.at[b0 + i]],
                            rows_v.at[i], gsem))
                    for i in range(GB):
                        gds[i].wait()
                        pltpu.async_copy(
                            rows_v.at[i], acc_sp.at[dst_v.at[b0 + i]],
                            ssem, add=True)
                    return carry

                lax.fori_loop(0, MC * MGROUPS, group, 0)
                drain_scatters()
                plsc.subcore_barrier()

                def wb_j(j, carry):
                    rr = r0 + j * ZR
                    pltpu.sync_copy(acc_sp.at[pl.ds(rr, ZR)], wb_v)
                    pltpu.sync_copy(wb_v, accs[slab].at[pl.ds(rr, ZR)])
                    return carry

                lax.fori_loop(0, NZ, wb_j, 0)

        @pl.when(c == 0)
        def _():
            process(hu_t, srcc_h, dstc_h, ai_t)

        @pl.when(c == 1)
        def _():
            process(hi_t, srcb_h, dstb_h, au_t)

    outs = sc_agg(*hu, *hi, src_c, dst_c, src_b, dst_b, zero_blk)
    return outs[0:NSLAB], outs[NSLAB:2 * NSLAB]


def _sc_count(dst_c, dst_b):
    zero_blk = jnp.zeros((ZC, CW), jnp.float32)
    ones_blk = jnp.ones((EB, CW), jnp.float32)
    cnt_ty = jax.ShapeDtypeStruct((NP, CW), jnp.float32)

    @functools.partial(
        pl.kernel,
        mesh=_MESH,
        out_type=[cnt_ty, cnt_ty],
        scratch_types=[
            pltpu.VMEM((CB, EB), jnp.int32),
            pltpu.VMEM((EB, CW), jnp.float32),
            pltpu.VMEM((ZC, CW), jnp.float32),
            pltpu.VMEM_SHARED((NP, CW), jnp.float32),
        ],
        compiler_params=_SC_PARAMS,
    )
    def sc_cnt(dstc_h, dstb_h, z_h, ones_h, co_i, co_u,
               dst_v, ones_v, z_v, cnt_sp):
        c = lax.axis_index("c")
        t = lax.axis_index("s")
        r0 = t * RPT

        def process(dst_h, cnt):
            pltpu.sync_copy(ones_h, ones_v)
            pltpu.sync_copy(z_h, z_v)
            for j in range(NZC):
                pltpu.sync_copy(z_v, cnt_sp.at[pl.ds(r0 + j * ZC, ZC)])
            plsc.subcore_barrier()

            def chunk(ci_, carry):
                pltpu.sync_copy(dst_h.at[t, ci_], dst_v)
                for b in range(CB):
                    pltpu.sync_copy(
                        ones_v, cnt_sp.at[dst_v.at[b]], add=True)
                return carry

            lax.fori_loop(0, NC, chunk, 0)
            plsc.subcore_barrier()
            for j in range(NZC):
                rr = r0 + j * ZC
                pltpu.sync_copy(cnt_sp.at[pl.ds(rr, ZC)], z_v)
                pltpu.sync_copy(z_v, cnt.at[pl.ds(rr, ZC)])

        @pl.when(c == 0)
        def _():
            process(dstc_h, co_i)

        @pl.when(c == 1)
        def _():
            process(dstb_h, co_u)

    return sc_cnt(dst_c, dst_b, zero_blk, ones_blk)


def _tc_combine(h_pad, accs, cnt, W_n, b_n, W_s, b_s):
    BLK = 512
    grid = (NP // BLK,)

    def body(*refs):
        h_ref = refs[0]
        a_refs = refs[1:1 + NSLAB]
        cnt_ref, wn_ref, bn_ref, ws_ref, bs_ref, o_ref = refs[1 + NSLAB:]
        cntv = cnt_ref[...][:, 0:1]
        inv = 1.0 / jnp.maximum(cntv, 1.0)
        a = jnp.concatenate([r[...] for r in a_refs], axis=1) * inv
        neigh = lax.dot_general(a, wn_ref[...], (((1,), (1,)), ((), ())),
                                preferred_element_type=jnp.float32)
        selfv = lax.dot_general(h_ref[...], ws_ref[...],
                                (((1,), (1,)), ((), ())),
                                preferred_element_type=jnp.float32)
        mask = (cntv > 0.0).astype(jnp.float32)
        o_ref[...] = neigh + mask * bn_ref[...] + selfv + bs_ref[...]

    row_spec = lambda w: pl.BlockSpec((BLK, w), lambda i: (i, 0))
    full_spec = lambda r, c: pl.BlockSpec((r, c), lambda i: (0, 0))
    return pl.pallas_call(
        body,
        grid=grid,
        in_specs=[row_spec(D)] + [row_spec(S)] * NSLAB + [row_spec(CW)]
        + [full_spec(D, D), full_spec(1, D), full_spec(D, D), full_spec(1, D)],
        out_specs=row_spec(D),
        out_shape=jax.ShapeDtypeStruct((NP, D), jnp.float32),
    )(h_pad, *accs, cnt, W_n, b_n, W_s, b_s)


def kernel(h_user, h_item, edge_index_clicks, edge_index_clicked_by,
           W_clicks, b_clicks, W_clicked_by, b_clicked_by,
           W_self_user, b_self_user, W_self_item, b_self_item):
    agg_shape = (TILES, MC, BPM, EB)
    cnt_shape = (TILES, NC, CB, EB)
    src_c, dst_c = _prep_edges(edge_index_clicks, E_PAD, agg_shape)
    src_b, dst_b = _prep_edges(edge_index_clicked_by, E_PAD, agg_shape)
    _, dst_c2 = _prep_edges(edge_index_clicks, EC_PAD, cnt_shape)
    _, dst_b2 = _prep_edges(edge_index_clicked_by, EC_PAD, cnt_shape)
    hu = _slabs(h_user)
    hi = _slabs(h_item)

    acc_item, acc_user = _sc_aggregate(hu, hi, src_c, dst_c, src_b, dst_b)
    cnt_item, cnt_user = _sc_count(dst_c2, dst_b2)

    pad = jnp.zeros((NP - N, D), jnp.float32)
    h_user_pad = jnp.concatenate([h_user, pad])
    h_item_pad = jnp.concatenate([h_item, pad])

    out_user = _tc_combine(h_user_pad, acc_user, cnt_user,
                           W_clicked_by, b_clicked_by.reshape(1, D),
                           W_self_user, b_self_user.reshape(1, D))
    out_item = _tc_combine(h_item_pad, acc_item, cnt_item,
                           W_clicks, b_clicks.reshape(1, D),
                           W_self_item, b_self_item.reshape(1, D))
    return (out_user[:N], out_item[:N])


# speedup vs baseline: 5.0743x; 1.3798x over previous
import functools

import jax
import jax.numpy as jnp
from jax import lax
from jax.experimental import pallas as pl
from jax.experimental.pallas import tpu as pltpu
from jax.experimental.pallas import tpu_sc as plsc

N = 50000
E = 500000
D = 128
S = 32
NSLAB = D // S
TILES = 16
EB = 128
GB = 4
BPM = 32
MC = 8
MGROUPS = BPM // GB
NB = MC * BPM
EPT = NB * EB
E_PAD = TILES * EPT
CB = 7
NC = 35
NBC = NC * CB
EPTC = NBC * EB
EC_PAD = TILES * EPTC
NP = 50176
RPT = NP // TILES
ZR = 112
NZ = RPT // ZR
CW = 8
ZC = 448
NZC = RPT // ZC


def _prep_edges(ei, e_pad, shape):
    src = ei[0].astype(jnp.int32)
    dst = ei[1].astype(jnp.int32)
    npad = e_pad - E
    ar = jnp.arange(npad, dtype=jnp.int32)
    src_p = jnp.concatenate([src, ar % N])
    dst_p = jnp.concatenate([dst, N + (ar % (NP - N))])
    return src_p.reshape(shape), dst_p.reshape(shape)


def _rowview(h):
    return h.reshape(N * NSLAB, S)


_MESH = plsc.VectorSubcoreMesh(core_axis_name="c", subcore_axis_name="s")
_SC_PARAMS = pltpu.CompilerParams(use_tc_tiling_on_sc=False)


def _sc_aggregate(hu, hi, src_c, dst_c, src_b, dst_b):
    zero_blk = jnp.zeros((ZR, S), jnp.float32)
    acc_ty = jax.ShapeDtypeStruct((NP, S), jnp.float32)

    @functools.partial(
        pl.kernel,
        mesh=_MESH,
        out_type=[acc_ty] * (2 * NSLAB),
        scratch_types=[
            pltpu.VMEM((BPM * EB,), jnp.int32),
            pltpu.VMEM((BPM, EB), jnp.int32),
            pltpu.VMEM((GB, EB, S), jnp.float32),
            pltpu.VMEM((ZR, S), jnp.float32),
            pltpu.VMEM_SHARED((NP, S), jnp.float32),
            pltpu.SemaphoreType.DMA,
            pltpu.SemaphoreType.DMA,
        ],
        compiler_params=_SC_PARAMS,
    )
    def sc_agg(*refs):
        hu_r, hi_r = refs[0:2]
        srcc_h, dstc_h, srcb_h, dstb_h, zb_h = refs[2:7]
        o = 7
        ai_t = refs[o:o + NSLAB]
        au_t = refs[o + NSLAB:o + 2 * NSLAB]
        (src_v, dst_v, rows_v, wb_v, acc_sp,
         gsem, ssem) = refs[o + 2 * NSLAB:]
        c = lax.axis_index("c")
        t = lax.axis_index("s")
        r0 = t * RPT

        def drain_scatters():
            for i in range(GB):
                pltpu.make_async_copy(
                    rows_v.at[i], acc_sp.at[dst_v.at[0]], ssem).wait()

        def process(table, src_h, dst_h, accs):
            for slab in range(NSLAB):
                pltpu.sync_copy(zb_h, wb_v)

                def zero_j(j, carry):
                    pltpu.sync_copy(wb_v, acc_sp.at[pl.ds(r0 + j * ZR, ZR)])
                    return carry

                lax.fori_loop(0, NZ, zero_j, 0)
                plsc.subcore_barrier()

                def group(g, carry):
                    @pl.when(g > 0)
                    def _():
                        drain_scatters()

                    @pl.when(lax.rem(g, MGROUPS) == 0)
                    def _():
                        m = lax.div(g, MGROUPS)
                        pltpu.sync_copy(src_h.at[t, m], src_v)
                        pltpu.sync_copy(dst_h.at[t, m], dst_v)

                        def xf(j, carry2):
                            sl = src_v[pl.ds(j * 16, 16)]
                            src_v[pl.ds(j * 16, 16)] = sl * NSLAB + slab
                            return carry2

                        lax.fori_loop(0, (BPM * EB) // 16, xf, 0)

                    b0 = lax.rem(g, MGROUPS) * GB
                    gds = []
                    for i in range(GB):
                        gds.append(pltpu.async_copy(
                            table.at[src_v.at[pl.ds((b0 + i) * EB, EB)]],
                            rows_v.at[i], gsem))
                    for i in range(GB):
                        gds[i].wait()
                        pltpu.async_copy(
                            rows_v.at[i], acc_sp.at[dst_v.at[b0 + i]],
                            ssem, add=True)
                    return carry

                lax.fori_loop(0, MC * MGROUPS, group, 0)
                drain_scatters()
                plsc.subcore_barrier()

                def wb_j(j, carry):
                    rr = r0 + j * ZR
                    pltpu.sync_copy(acc_sp.at[pl.ds(rr, ZR)], wb_v)
                    pltpu.sync_copy(wb_v, accs[slab].at[pl.ds(rr, ZR)])
                    return carry

                lax.fori_loop(0, NZ, wb_j, 0)

        @pl.when(c == 0)
        def _():
            process(hu_r, srcc_h, dstc_h, ai_t)

        @pl.when(c == 1)
        def _():
            process(hi_r, srcb_h, dstb_h, au_t)

    outs = sc_agg(hu, hi, src_c, dst_c, src_b, dst_b, zero_blk)
    return outs[0:NSLAB], outs[NSLAB:2 * NSLAB]


def _sc_count(dst_c, dst_b):
    zero_blk = jnp.zeros((ZC, CW), jnp.float32)
    ones_blk = jnp.ones((EB, CW), jnp.float32)
    cnt_ty = jax.ShapeDtypeStruct((NP, CW), jnp.float32)

    @functools.partial(
        pl.kernel,
        mesh=_MESH,
        out_type=[cnt_ty, cnt_ty],
        scratch_types=[
            pltpu.VMEM((BPM, EB), jnp.int32),
            pltpu.VMEM((EB, CW), jnp.float32),
            pltpu.VMEM((ZC, CW), jnp.float32),
            pltpu.VMEM_SHARED((NP, CW), jnp.float32),
        ],
        compiler_params=_SC_PARAMS,
    )
    def sc_cnt(dstc_h, dstb_h, z_h, ones_h, co_i, co_u,
               dst_v, ones_v, z_v, cnt_sp):
        c = lax.axis_index("c")
        t = lax.axis_index("s")
        r0 = t * RPT

        def process(dst_h, cnt):
            pltpu.sync_copy(ones_h, ones_v)
            pltpu.sync_copy(z_h, z_v)
            for j in range(NZC):
                pltpu.sync_copy(z_v, cnt_sp.at[pl.ds(r0 + j * ZC, ZC)])
            plsc.subcore_barrier()

            def chunk(m, carry):
                pltpu.sync_copy(dst_h.at[t, m], dst_v)

                def bat(b, carry2):
                    pltpu.sync_copy(
                        ones_v, cnt_sp.at[dst_v.at[b]], add=True)
                    return carry2

                lax.fori_loop(0, BPM, bat, 0)
                return carry

            lax.fori_loop(0, MC, chunk, 0)
            plsc.subcore_barrier()
            for j in range(NZC):
                rr = r0 + j * ZC
                pltpu.sync_copy(cnt_sp.at[pl.ds(rr, ZC)], z_v)
                pltpu.sync_copy(z_v, cnt.at[pl.ds(rr, ZC)])

        @pl.when(c == 0)
        def _():
            process(dstc_h, co_i)

        @pl.when(c == 1)
        def _():
            process(dstb_h, co_u)

    return sc_cnt(dst_c, dst_b, zero_blk, ones_blk)


TBLK = 1000


def _tc_linear(h, W, b):
    def body(h_ref, w_ref, b_ref, o_ref):
        o_ref[...] = lax.dot_general(
            h_ref[...], w_ref[...], (((1,), (1,)), ((), ())),
            preferred_element_type=jnp.float32) + b_ref[...]

    row_spec = lambda w: pl.BlockSpec((TBLK, w), lambda i: (i, 0))
    full_spec = lambda r, c: pl.BlockSpec((r, c), lambda i: (0, 0))
    return pl.pallas_call(
        body,
        grid=(N // TBLK,),
        in_specs=[row_spec(D), full_spec(D, D), full_spec(1, D)],
        out_specs=row_spec(D),
        out_shape=jax.ShapeDtypeStruct((N, D), jnp.float32),
    )(h, W, b)


def _tc_combine(selfv, accs, cnt, W_n, b_n):
    def body(*refs):
        s_ref = refs[0]
        a_refs = refs[1:1 + NSLAB]
        cnt_ref, wn_ref, bn_ref, o_ref = refs[1 + NSLAB:]
        cntv = cnt_ref[...][:, 0:1]
        inv = 1.0 / jnp.maximum(cntv, 1.0)
        a = jnp.concatenate([r[...] for r in a_refs], axis=1) * inv
        neigh = lax.dot_general(a, wn_ref[...], (((1,), (1,)), ((), ())),
                                preferred_element_type=jnp.float32)
        mask = (cntv > 0.0).astype(jnp.float32)
        o_ref[...] = neigh + mask * bn_ref[...] + s_ref[...]

    row_spec = lambda w: pl.BlockSpec((TBLK, w), lambda i: (i, 0))
    full_spec = lambda r, c: pl.BlockSpec((r, c), lambda i: (0, 0))
    return pl.pallas_call(
        body,
        grid=(N // TBLK,),
        in_specs=[row_spec(D)] + [row_spec(S)] * NSLAB + [row_spec(CW)]
        + [full_spec(D, D), full_spec(1, D)],
        out_specs=row_spec(D),
        out_shape=jax.ShapeDtypeStruct((N, D), jnp.float32),
    )(selfv, *accs, cnt, W_n, b_n)


def kernel(h_user, h_item, edge_index_clicks, edge_index_clicked_by,
           W_clicks, b_clicks, W_clicked_by, b_clicked_by,
           W_self_user, b_self_user, W_self_item, b_self_item):
    agg_shape = (TILES, MC, BPM, EB)
    src_c, dst_c = _prep_edges(edge_index_clicks, E_PAD, agg_shape)
    src_b, dst_b = _prep_edges(edge_index_clicked_by, E_PAD, agg_shape)
    src_c = src_c.reshape(TILES, MC, BPM * EB)
    src_b = src_b.reshape(TILES, MC, BPM * EB)
    hu = _rowview(h_user)
    hi = _rowview(h_item)

    self_user = _tc_linear(h_user, W_self_user, b_self_user.reshape(1, D))
    self_item = _tc_linear(h_item, W_self_item, b_self_item.reshape(1, D))

    acc_item, acc_user = _sc_aggregate(hu, hi, src_c, dst_c, src_b, dst_b)
    cnt_item, cnt_user = _sc_count(dst_c, dst_b)

    out_user = _tc_combine(self_user, acc_user, cnt_user,
                           W_clicked_by, b_clicked_by.reshape(1, D))
    out_item = _tc_combine(self_item, acc_item, cnt_item,
                           W_clicks, b_clicks.reshape(1, D))
    return (out_user, out_item)
